# baseline (speedup 1.0000x reference)
import jax
import jax.numpy as jnp
from jax.experimental import pallas as pl
from jax.experimental.pallas import tpu as pltpu

_N = 4194304
_COLS = 128
_ROWS = _N // _COLS
_BR = 4096
_GRID = _ROWS // _BR


def _tc_body(y_ref, s_ref, out_ref, acc_ref):
    i = pl.program_id(0)
    yb = y_ref[...]
    sf = s_ref[...].astype(jnp.float32)
    tot = jnp.sum(yb.reshape(_BR // 8, 8, _COLS), axis=0)
    s1 = jnp.sum((yb * sf).reshape(_BR // 8, 8, _COLS), axis=0)
    c1 = jnp.sum(sf.reshape(_BR // 8, 8, _COLS), axis=0)

    @pl.when(i == 0)
    def _init():
        acc_ref[0] = tot
        acc_ref[1] = s1
        acc_ref[2] = c1

    @pl.when(i > 0)
    def _acc():
        acc_ref[0] += tot
        acc_ref[1] += s1
        acc_ref[2] += c1

    @pl.when(i == _GRID - 1)
    def _finalize():
        total = jnp.sum(acc_ref[0])
        sum1 = jnp.sum(acc_ref[1])
        c1t = jnp.sum(acc_ref[2])
        c0t = jnp.float32(_N) - c1t
        sum0 = total - sum1
        out_ref[0, 0] = jnp.abs(sum0 / c0t - sum1 / c1t)


_tc_reduce = pl.pallas_call(
    _tc_body,
    grid=(_GRID,),
    in_specs=[
        pl.BlockSpec((_BR, _COLS), lambda i: (i, 0)),
        pl.BlockSpec((_BR, _COLS), lambda i: (i, 0)),
    ],
    out_specs=pl.BlockSpec(memory_space=pltpu.SMEM),
    out_shape=jax.ShapeDtypeStruct((1, 1), jnp.float32),
    scratch_shapes=[pltpu.VMEM((3, 8, _COLS), jnp.float32)],
    compiler_params=pltpu.CompilerParams(
        dimension_semantics=("arbitrary",),
    ),
)


def kernel(y_pred, s, y_gt):
    del y_gt
    y2 = y_pred.reshape(_ROWS, _COLS)
    s2 = s.reshape(_ROWS, _COLS)
    reg_loss = _tc_reduce(y2, s2)[0, 0]
    zero = jnp.zeros((1,), dtype=jnp.float32)
    return (reg_loss, zero, zero, zero)


# speedup vs baseline: 1.0104x; 1.0104x over previous
import jax
import jax.numpy as jnp
from jax.experimental import pallas as pl
from jax.experimental.pallas import tpu as pltpu

_N = 4194304
_COLS = 128
_ROWS = _N // _COLS
_BR = 16384
_GRID = _ROWS // _BR


def _tc_body(y_ref, s_ref, out_ref, acc_ref):
    i = pl.program_id(0)
    yb = y_ref[...]
    sf = s_ref[...].astype(jnp.float32)
    tot = jnp.sum(yb.reshape(_BR // 8, 8, _COLS), axis=0)
    s1 = jnp.sum((yb * sf).reshape(_BR // 8, 8, _COLS), axis=0)
    c1 = jnp.sum(sf.reshape(_BR // 8, 8, _COLS), axis=0)

    @pl.when(i == 0)
    def _init():
        acc_ref[0] = tot
        acc_ref[1] = s1
        acc_ref[2] = c1

    @pl.when(i > 0)
    def _acc():
        acc_ref[0] += tot
        acc_ref[1] += s1
        acc_ref[2] += c1

    @pl.when(i == _GRID - 1)
    def _finalize():
        total = jnp.sum(acc_ref[0])
        sum1 = jnp.sum(acc_ref[1])
        c1t = jnp.sum(acc_ref[2])
        c0t = jnp.float32(_N) - c1t
        sum0 = total - sum1
        out_ref[0, 0] = jnp.abs(sum0 / c0t - sum1 / c1t)


_tc_reduce = pl.pallas_call(
    _tc_body,
    grid=(_GRID,),
    in_specs=[
        pl.BlockSpec((_BR, _COLS), lambda i: (i, 0)),
        pl.BlockSpec((_BR, _COLS), lambda i: (i, 0)),
    ],
    out_specs=pl.BlockSpec(memory_space=pltpu.SMEM),
    out_shape=jax.ShapeDtypeStruct((1, 1), jnp.float32),
    scratch_shapes=[pltpu.VMEM((3, 8, _COLS), jnp.float32)],
    compiler_params=pltpu.CompilerParams(
        dimension_semantics=("arbitrary",),
    ),
)


def kernel(y_pred, s, y_gt):
    del y_gt
    y2 = y_pred.reshape(_ROWS, _COLS)
    s2 = s.reshape(_ROWS, _COLS)
    reg_loss = _tc_reduce(y2, s2)[0, 0]
    zero = jnp.zeros((1,), dtype=jnp.float32)
    return (reg_loss, zero, zero, zero)


# speedup vs baseline: 1.0625x; 1.0516x over previous
import jax
import jax.numpy as jnp
from jax.experimental import pallas as pl
from jax.experimental.pallas import tpu as pltpu

_N = 4194304
_COLS = 128
_ROWS = _N // _COLS
_BR = 8192
_GRID = _ROWS // _BR


def _tc_body(y_ref, s_ref, out_ref, acc_ref):
    i = pl.program_id(0)
    yb = y_ref[...]
    sf = s_ref[...].astype(jnp.float32)
    tot = jnp.sum(yb.reshape(_BR // 8, 8, _COLS), axis=0)
    s1 = jnp.sum((yb * sf).reshape(_BR // 8, 8, _COLS), axis=0)
    c1 = jnp.sum(sf.reshape(_BR // 8, 8, _COLS), axis=0)

    @pl.when(i == 0)
    def _init():
        acc_ref[0] = tot
        acc_ref[1] = s1
        acc_ref[2] = c1

    @pl.when(i > 0)
    def _acc():
        acc_ref[0] += tot
        acc_ref[1] += s1
        acc_ref[2] += c1

    @pl.when(i == _GRID - 1)
    def _finalize():
        total = jnp.sum(acc_ref[0])
        sum1 = jnp.sum(acc_ref[1])
        c1t = jnp.sum(acc_ref[2])
        c0t = jnp.float32(_N) - c1t
        sum0 = total - sum1
        out_ref[0, 0] = jnp.abs(sum0 / c0t - sum1 / c1t)


_tc_reduce = pl.pallas_call(
    _tc_body,
    grid=(_GRID,),
    in_specs=[
        pl.BlockSpec((_BR, _COLS), lambda i: (i, 0)),
        pl.BlockSpec((_BR, _COLS), lambda i: (i, 0)),
    ],
    out_specs=pl.BlockSpec(memory_space=pltpu.SMEM),
    out_shape=jax.ShapeDtypeStruct((1, 1), jnp.float32),
    scratch_shapes=[pltpu.VMEM((3, 8, _COLS), jnp.float32)],
    compiler_params=pltpu.CompilerParams(
        dimension_semantics=("arbitrary",),
    ),
)


def kernel(y_pred, s, y_gt):
    del y_gt
    y2 = y_pred.reshape(_ROWS, _COLS)
    s2 = s.reshape(_ROWS, _COLS)
    reg_loss = _tc_reduce(y2, s2)[0, 0]
    zero = jnp.zeros((1,), dtype=jnp.float32)
    return (reg_loss, zero, zero, zero)


# speedup vs baseline: 1.1475x; 1.0799x over previous
import jax
import jax.numpy as jnp
from jax.experimental import pallas as pl
from jax.experimental.pallas import tpu as pltpu

_N = 4194304
_COLS = 128
_ROWS = _N // _COLS
_CH = 2048
_NCH = _ROWS // _CH
_DEPTH = 4


def _tc_body(y_hbm, s_hbm, out_ref, ybuf, sbuf, *sems):
    def start(c):
        slot = c % _DEPTH
        pltpu.async_copy(y_hbm.at[pl.ds(c * _CH, _CH)], ybuf.at[slot],
                         sems[slot])
        pltpu.async_copy(s_hbm.at[pl.ds(c * _CH, _CH)], sbuf.at[slot],
                         sems[slot])

    def wait(c):
        slot = c % _DEPTH
        pltpu.make_async_copy(y_hbm.at[pl.ds(0, _CH)], ybuf.at[slot],
                              sems[slot]).wait()
        pltpu.make_async_copy(s_hbm.at[pl.ds(0, _CH)], sbuf.at[slot],
                              sems[slot]).wait()

    for c in range(_DEPTH):
        start(c)

    acc = None
    for c in range(_NCH):
        slot = c % _DEPTH
        wait(c)
        yb = ybuf[slot]
        sf = sbuf[slot].astype(jnp.float32)
        tot = jnp.sum(yb.reshape(_CH // 8, 8, _COLS), axis=0)
        s1 = jnp.sum((yb * sf).reshape(_CH // 8, 8, _COLS), axis=0)
        c1 = jnp.sum(sf.reshape(_CH // 8, 8, _COLS), axis=0)
        if acc is None:
            acc = [tot, s1, c1]
        else:
            acc = [acc[0] + tot, acc[1] + s1, acc[2] + c1]
        if c + _DEPTH < _NCH:
            start(c + _DEPTH)

    total = jnp.sum(acc[0])
    sum1 = jnp.sum(acc[1])
    c1t = jnp.sum(acc[2])
    c0t = jnp.float32(_N) - c1t
    sum0 = total - sum1
    out_ref[0, 0] = jnp.abs(sum0 / c0t - sum1 / c1t)


_tc_reduce = pl.pallas_call(
    _tc_body,
    in_specs=[
        pl.BlockSpec(memory_space=pl.ANY),
        pl.BlockSpec(memory_space=pl.ANY),
    ],
    out_specs=pl.BlockSpec(memory_space=pltpu.SMEM),
    out_shape=jax.ShapeDtypeStruct((1, 1), jnp.float32),
    scratch_shapes=(
        [pltpu.VMEM((_DEPTH, _CH, _COLS), jnp.float32),
         pltpu.VMEM((_DEPTH, _CH, _COLS), jnp.int32)]
        + [pltpu.SemaphoreType.DMA] * _DEPTH
    ),
)


def kernel(y_pred, s, y_gt):
    del y_gt
    y2 = y_pred.reshape(_ROWS, _COLS)
    s2 = s.reshape(_ROWS, _COLS)
    reg_loss = _tc_reduce(y2, s2)[0, 0]
    zero = jnp.zeros((1,), dtype=jnp.float32)
    return (reg_loss, zero, zero, zero)
